# Initial kernel scaffold; baseline (speedup 1.0000x reference)
import jax
import jax.numpy as jnp
from jax.experimental import pallas as pl


def kernel(edge_index, node_norm, user_emb, item_emb, bn_gamma, bn_beta):
    raise NotImplementedError("write your pallas kernel here")


# speedup vs baseline: 1.6119x; 1.6119x over previous
import jax
import jax.numpy as jnp
from jax.experimental import pallas as pl

_EPS = 1e-5
_N_USERS = 5000


def _bn_body(h_ref, g_ref, b_ref, acc_ref, hout_ref, accout_ref):
    h = h_ref[...]
    mean = jnp.mean(h, axis=0, keepdims=True)
    var = jnp.mean((h - mean) ** 2, axis=0, keepdims=True)
    hn = (h - mean) * jax.lax.rsqrt(var + _EPS) * g_ref[...] + b_ref[...]
    hr = jnp.maximum(hn, 0.0)
    hout_ref[...] = hr
    accout_ref[...] = acc_ref[...] + hr


def _bn_step(h_raw, gamma, beta, acc):
    n, d = h_raw.shape
    return pl.pallas_call(
        _bn_body,
        out_shape=(
            jax.ShapeDtypeStruct((n, d), h_raw.dtype),
            jax.ShapeDtypeStruct((n, d), h_raw.dtype),
        ),
    )(h_raw, gamma.reshape(1, d), beta.reshape(1, d), acc)


def kernel(edge_index, node_norm, user_emb, item_emb, bn_gamma, bn_beta):
    row, col = edge_index[0], edge_index[1]
    x = jnp.concatenate([user_emb, item_emb], axis=0)
    n = x.shape[0]
    deg = jax.ops.segment_sum(jnp.ones(row.shape, x.dtype), col, num_segments=n)
    dis = jnp.where(deg > 0, jax.lax.rsqrt(deg), 0.0)
    acc = x
    h = x
    for i in range(3):
        s = jax.ops.segment_sum((dis[:, None] * h)[row], col, num_segments=n)
        h_raw = dis[:, None] * s
        h, acc = _bn_step(h_raw, bn_gamma[i], bn_beta[i], acc)
    final = acc * 0.25 * node_norm[:, None]
    return final[:_N_USERS], final[_N_USERS:]


# speedup vs baseline: 5.0171x; 3.1125x over previous
import jax
import jax.numpy as jnp
from jax import lax
from jax.experimental import pallas as pl
from jax.experimental.pallas import tpu as pltpu
from jax.experimental.pallas import tpu_sc as plsc

N = 10000
NP = 10240
EMB = 256
H = 128
E = 160000
LAYERS = 3
EPS = 1e-5
N_USERS = 5000

NTILES = 16
CH = 128
NCH = NP // CH
EP = NTILES * NP
SLAB = NP // NTILES

def _mesh():
    return plsc.VectorSubcoreMesh(core_axis_name="c", subcore_axis_name="s")


def _deg_body(col_hbm, zeros_hbm, ones_hbm, deg_hbm, deg_sh, cb, ones):
    cid = lax.axis_index("c")
    tid = lax.axis_index("s")

    @pl.when(cid == 0)
    def _():
        slab = pl.ds(tid * SLAB, SLAB)
        pltpu.sync_copy(zeros_hbm.at[slab], deg_sh.at[slab])
        pltpu.sync_copy(ones_hbm, ones)
        cols = col_hbm.at[tid]
        plsc.subcore_barrier()

        @pl.loop(0, NCH)
        def _(k):
            pltpu.sync_copy(cols.at[k], cb)
            pltpu.sync_copy(ones, deg_sh.at[cb.at[0]], add=True)

        plsc.subcore_barrier()
        pltpu.sync_copy(deg_sh.at[slab], deg_hbm.at[slab])


def _deg_call(col4d, zeros128, ones128):
    return pl.kernel(
        _deg_body,
        out_type=jax.ShapeDtypeStruct((NP, H), jnp.float32),
        mesh=_mesh(),
        scratch_types=[
            pltpu.VMEM_SHARED((NP, H), jnp.float32),
            pltpu.VMEM((1, CH), jnp.int32),
            pltpu.VMEM((CH, H), jnp.float32),
        ],
    )(col4d, zeros128, ones128)


def _conv_body(y_hbm, row_hbm, col_hbm, zeros_hbm, s_hbm,
               acc_sh, r0, r1, c0, c1, msg0, msg1, sem0, sem1):
    cid = lax.axis_index("c")
    tid = lax.axis_index("s")
    slab = pl.ds(tid * SLAB, SLAB)

    pltpu.sync_copy(zeros_hbm.at[slab], acc_sh.at[slab])
    plsc.subcore_barrier()

    ysrc = y_hbm.at[cid]
    rows = row_hbm.at[tid]
    cols = col_hbm.at[tid]

    pltpu.sync_copy(rows.at[0], r0)
    pltpu.sync_copy(cols.at[0], c0)
    pltpu.make_async_copy(ysrc.at[r0.at[0]], msg0, sem0).start()
    pltpu.sync_copy(rows.at[1], r1)
    pltpu.sync_copy(cols.at[1], c1)
    pltpu.make_async_copy(ysrc.at[r1.at[0]], msg1, sem1).start()

    @pl.loop(0, NCH, step=2)
    def _(k):
        pltpu.make_async_copy(ysrc.at[r0.at[0]], msg0, sem0).wait()
        pltpu.sync_copy(msg0, acc_sh.at[c0.at[0]], add=True)

        @pl.when(k + 2 < NCH)
        def _():
            pltpu.sync_copy(rows.at[k + 2], r0)
            pltpu.sync_copy(cols.at[k + 2], c0)
            pltpu.make_async_copy(ysrc.at[r0.at[0]], msg0, sem0).start()

        pltpu.make_async_copy(ysrc.at[r1.at[0]], msg1, sem1).wait()
        pltpu.sync_copy(msg1, acc_sh.at[c1.at[0]], add=True)

        @pl.when(k + 3 < NCH)
        def _():
            pltpu.sync_copy(rows.at[k + 3], r1)
            pltpu.sync_copy(cols.at[k + 3], c1)
            pltpu.make_async_copy(ysrc.at[r1.at[0]], msg1, sem1).start()

    plsc.subcore_barrier()
    pltpu.sync_copy(acc_sh.at[slab], s_hbm.at[cid].at[slab])


def _conv_call(y, row4d, col4d, zeros128):
    return pl.kernel(
        _conv_body,
        out_type=jax.ShapeDtypeStruct((2, NP, H), jnp.float32),
        mesh=_mesh(),
        scratch_types=[
            pltpu.VMEM_SHARED((NP, H), jnp.float32),
            pltpu.VMEM((1, CH), jnp.int32),
            pltpu.VMEM((1, CH), jnp.int32),
            pltpu.VMEM((1, CH), jnp.int32),
            pltpu.VMEM((1, CH), jnp.int32),
            pltpu.VMEM((CH, H), jnp.float32),
            pltpu.VMEM((CH, H), jnp.float32),
            pltpu.SemaphoreType.DMA,
            pltpu.SemaphoreType.DMA,
        ],
    )(y, row4d, col4d, zeros128)


def _prep_body(u_ref, i_ref, degb_ref, y_ref, dis_ref, xh_ref):
    deg = degb_ref[...]
    rid = lax.broadcasted_iota(jnp.int32, (NP, H), 0)
    dis = jnp.where((deg > 0) & (rid < N), lax.rsqrt(deg), 0.0)
    dis_ref[...] = dis
    for c in range(2):
        xc = jnp.concatenate(
            [u_ref[:, c * H:(c + 1) * H],
             i_ref[:, c * H:(c + 1) * H],
             jnp.zeros((NP - N, H), jnp.float32)], axis=0)
        xh_ref[c] = xc
        y_ref[c] = dis * xc


def _prep_call(user_emb, item_emb, deg_b):
    return pl.pallas_call(
        _prep_body,
        out_shape=(
            jax.ShapeDtypeStruct((2, NP, H), jnp.float32),
            jax.ShapeDtypeStruct((NP, H), jnp.float32),
            jax.ShapeDtypeStruct((2, NP, H), jnp.float32),
        ),
    )(user_emb, item_emb, deg_b)


def _bn_mid_body(s_ref, dis_ref, g_ref, b_ref, acc_ref, accout_ref, y_ref):
    dis = dis_ref[...]
    inv_n = jnp.float32(1.0 / N)
    for c in range(2):
        h = dis * s_ref[c]
        mean = jnp.sum(h, axis=0, keepdims=True) * inv_n
        var = jnp.sum(h * h, axis=0, keepdims=True) * inv_n - mean * mean
        hr = jnp.maximum(
            (h - mean) * lax.rsqrt(var + EPS) * g_ref[c] + b_ref[c], 0.0)
        accout_ref[c] = acc_ref[c] + hr
        y_ref[c] = dis * hr


def _bn_mid_call(s, dis, g, b, acc):
    return pl.pallas_call(
        _bn_mid_body,
        out_shape=(
            jax.ShapeDtypeStruct((2, NP, H), jnp.float32),
            jax.ShapeDtypeStruct((2, NP, H), jnp.float32),
        ),
    )(s, dis, g, b, acc)


def _bn_final_body(s_ref, dis_ref, g_ref, b_ref, acc_ref, nn_ref, f_ref):
    dis = dis_ref[...]
    nn = nn_ref[...]
    inv_n = jnp.float32(1.0 / N)
    for c in range(2):
        h = dis * s_ref[c]
        mean = jnp.sum(h, axis=0, keepdims=True) * inv_n
        var = jnp.sum(h * h, axis=0, keepdims=True) * inv_n - mean * mean
        hr = jnp.maximum(
            (h - mean) * lax.rsqrt(var + EPS) * g_ref[c] + b_ref[c], 0.0)
        f_ref[c] = (acc_ref[c] + hr) * 0.25 * nn


def _bn_final_call(s, dis, g, b, acc, nn_b):
    return pl.pallas_call(
        _bn_final_body,
        out_shape=jax.ShapeDtypeStruct((2, NP, H), jnp.float32),
    )(s, dis, g, b, acc, nn_b)


def kernel(edge_index, node_norm, user_emb, item_emb, bn_gamma, bn_beta):
    row = edge_index[0]
    col = edge_index[1]
    pad = jnp.full((EP - E,), NP - 1, jnp.int32)
    row4d = jnp.concatenate([row, pad]).reshape(NTILES, NCH, 1, CH)
    col_p = jnp.concatenate([col, pad])
    col4d = col_p.reshape(NTILES, NCH, 1, CH)
    zeros128 = jnp.zeros((NP, H), jnp.float32)
    ones128 = jnp.ones((CH, H), jnp.float32)

    deg_b = _deg_call(col4d, zeros128, ones128)
    y, dis, acc = _prep_call(user_emb, item_emb, deg_b)

    g = bn_gamma.reshape(LAYERS, 2, 1, H)
    b = bn_beta.reshape(LAYERS, 2, 1, H)
    nn_b = jnp.broadcast_to(
        jnp.pad(node_norm, (0, NP - N))[:, None], (NP, H))

    for i in range(LAYERS):
        s = _conv_call(y, row4d, col4d, zeros128)
        if i < LAYERS - 1:
            acc, y = _bn_mid_call(s, dis, g[i], b[i], acc)
        else:
            f = _bn_final_call(s, dis, g[i], b[i], acc, nn_b)

    final = jnp.concatenate([f[0], f[1]], axis=1)
    return final[:N_USERS], final[N_USERS:N]


# speedup vs baseline: 5.0209x; 1.0008x over previous
import jax
import jax.numpy as jnp
from jax import lax
from jax.experimental import pallas as pl
from jax.experimental.pallas import tpu as pltpu
from jax.experimental.pallas import tpu_sc as plsc

N = 10000
NP = 10240
EMB = 256
H = 128
E = 160000
LAYERS = 3
EPS = 1e-5
N_USERS = 5000

NTILES = 16
CH = 128
NCH = NP // CH
EP = NTILES * NP
SLAB = NP // NTILES

def _mesh():
    return plsc.VectorSubcoreMesh(core_axis_name="c", subcore_axis_name="s")


def _deg_body(col_hbm, zeros_hbm, ones_hbm, deg_hbm, deg_sh, colb, ones,
              s0, s1, s2, s3):
    cid = lax.axis_index("c")
    tid = lax.axis_index("s")
    ssem = (s0, s1, s2, s3)

    @pl.when(cid == 0)
    def _():
        slab = pl.ds(tid * SLAB, SLAB)
        pltpu.sync_copy(zeros_hbm.at[slab], deg_sh.at[slab])
        pltpu.sync_copy(ones_hbm, ones)
        pltpu.sync_copy(col_hbm.at[tid], colb)
        plsc.subcore_barrier()

        @pl.loop(0, NCH, step=4)
        def _(k0):
            for b in range(4):
                k = k0 + b

                @pl.when(k >= 4)
                def _():
                    pltpu.make_async_copy(
                        ones, deg_sh.at[colb.at[k - 4]], ssem[b]).wait()
                pltpu.make_async_copy(
                    ones, deg_sh.at[colb.at[k]], ssem[b]).start(add=True)

        for b in range(4):
            pltpu.make_async_copy(
                ones, deg_sh.at[colb.at[NCH - 4 + b]], ssem[b]).wait()

        plsc.subcore_barrier()
        pltpu.sync_copy(deg_sh.at[slab], deg_hbm.at[slab])


def _deg_call(col3d, zeros128, ones128):
    return pl.kernel(
        _deg_body,
        out_type=jax.ShapeDtypeStruct((NP, H), jnp.float32),
        mesh=_mesh(),
        scratch_types=[
            pltpu.VMEM_SHARED((NP, H), jnp.float32),
            pltpu.VMEM((NCH, CH), jnp.int32),
            pltpu.VMEM((CH, H), jnp.float32),
            pltpu.SemaphoreType.DMA,
            pltpu.SemaphoreType.DMA,
            pltpu.SemaphoreType.DMA,
            pltpu.SemaphoreType.DMA,
        ],
    )(col3d, zeros128, ones128)


def _conv_body(y_hbm, rc_hbm, zeros_hbm, s_hbm,
               acc_sh, rc0, rc1, m0, m1,
               i0, i1, g0, g1, s0, s1):
    cid = lax.axis_index("c")
    tid = lax.axis_index("s")
    slab = pl.ds(tid * SLAB, SLAB)
    rcb = (rc0, rc1)
    msgs = (m0, m1)
    isem = (i0, i1)
    gsem = (g0, g1)
    ssem = (s0, s1)

    pltpu.sync_copy(zeros_hbm.at[slab], acc_sh.at[slab])
    plsc.subcore_barrier()

    ysrc = y_hbm.at[cid]
    rcs = rc_hbm.at[tid]

    @pl.loop(0, NCH, step=2)
    def _(k0):
        for b in range(2):
            k = k0 + b
            o = b ^ 1

            @pl.when(k >= 2)
            def _():
                pltpu.make_async_copy(msgs[b], acc_sh.at[rcb[b].at[1]],
                                      ssem[b]).wait()
            pltpu.make_async_copy(rcs.at[k], rcb[b], isem[b]).start()

            @pl.when(k >= 1)
            def _():
                pltpu.make_async_copy(ysrc.at[rcb[o].at[0]], msgs[o],
                                      gsem[o]).wait()
                pltpu.make_async_copy(msgs[o], acc_sh.at[rcb[o].at[1]],
                                      ssem[o]).start(add=True)
            pltpu.make_async_copy(rcs.at[k], rcb[b], isem[b]).wait()
            pltpu.make_async_copy(ysrc.at[rcb[b].at[0]], msgs[b],
                                  gsem[b]).start()

    lb = (NCH - 1) % 2
    pltpu.make_async_copy(ysrc.at[rcb[lb].at[0]], msgs[lb], gsem[lb]).wait()
    pltpu.make_async_copy(msgs[lb], acc_sh.at[rcb[lb].at[1]],
                          ssem[lb]).start(add=True)
    pltpu.make_async_copy(msgs[0], acc_sh.at[rcb[0].at[1]], ssem[0]).wait()
    pltpu.make_async_copy(msgs[1], acc_sh.at[rcb[1].at[1]], ssem[1]).wait()

    plsc.subcore_barrier()
    pltpu.sync_copy(acc_sh.at[slab], s_hbm.at[cid].at[slab])


def _conv_call(y, rc, zeros128):
    return pl.kernel(
        _conv_body,
        out_type=jax.ShapeDtypeStruct((2, NP, H), jnp.float32),
        mesh=_mesh(),
        scratch_types=[
            pltpu.VMEM_SHARED((NP, H), jnp.float32),
            pltpu.VMEM((2, CH), jnp.int32),
            pltpu.VMEM((2, CH), jnp.int32),
            pltpu.VMEM((CH, H), jnp.float32),
            pltpu.VMEM((CH, H), jnp.float32),
            pltpu.SemaphoreType.DMA,
            pltpu.SemaphoreType.DMA,
            pltpu.SemaphoreType.DMA,
            pltpu.SemaphoreType.DMA,
            pltpu.SemaphoreType.DMA,
            pltpu.SemaphoreType.DMA,
        ],
    )(y, rc, zeros128)


def _prep_body(u_ref, i_ref, degb_ref, y_ref, dis_ref, xh_ref):
    deg = degb_ref[...]
    rid = lax.broadcasted_iota(jnp.int32, (NP, H), 0)
    dis = jnp.where((deg > 0) & (rid < N), lax.rsqrt(deg), 0.0)
    dis_ref[...] = dis
    for c in range(2):
        xc = jnp.concatenate(
            [u_ref[:, c * H:(c + 1) * H],
             i_ref[:, c * H:(c + 1) * H],
             jnp.zeros((NP - N, H), jnp.float32)], axis=0)
        xh_ref[c] = xc
        y_ref[c] = dis * xc


def _prep_call(user_emb, item_emb, deg_b):
    return pl.pallas_call(
        _prep_body,
        out_shape=(
            jax.ShapeDtypeStruct((2, NP, H), jnp.float32),
            jax.ShapeDtypeStruct((NP, H), jnp.float32),
            jax.ShapeDtypeStruct((2, NP, H), jnp.float32),
        ),
    )(user_emb, item_emb, deg_b)


def _bn_mid_body(s_ref, dis_ref, g_ref, b_ref, acc_ref, accout_ref, y_ref):
    dis = dis_ref[...]
    inv_n = jnp.float32(1.0 / N)
    for c in range(2):
        h = dis * s_ref[c]
        mean = jnp.sum(h, axis=0, keepdims=True) * inv_n
        var = jnp.sum(h * h, axis=0, keepdims=True) * inv_n - mean * mean
        hr = jnp.maximum(
            (h - mean) * lax.rsqrt(var + EPS) * g_ref[c] + b_ref[c], 0.0)
        accout_ref[c] = acc_ref[c] + hr
        y_ref[c] = dis * hr


def _bn_mid_call(s, dis, g, b, acc):
    return pl.pallas_call(
        _bn_mid_body,
        out_shape=(
            jax.ShapeDtypeStruct((2, NP, H), jnp.float32),
            jax.ShapeDtypeStruct((2, NP, H), jnp.float32),
        ),
    )(s, dis, g, b, acc)


def _bn_final_body(s_ref, dis_ref, g_ref, b_ref, acc_ref, nn_ref, f_ref):
    dis = dis_ref[...]
    nn = nn_ref[...]
    inv_n = jnp.float32(1.0 / N)
    for c in range(2):
        h = dis * s_ref[c]
        mean = jnp.sum(h, axis=0, keepdims=True) * inv_n
        var = jnp.sum(h * h, axis=0, keepdims=True) * inv_n - mean * mean
        hr = jnp.maximum(
            (h - mean) * lax.rsqrt(var + EPS) * g_ref[c] + b_ref[c], 0.0)
        f_ref[c] = (acc_ref[c] + hr) * 0.25 * nn


def _bn_final_call(s, dis, g, b, acc, nn_b):
    return pl.pallas_call(
        _bn_final_body,
        out_shape=jax.ShapeDtypeStruct((2, NP, H), jnp.float32),
    )(s, dis, g, b, acc, nn_b)


def kernel(edge_index, node_norm, user_emb, item_emb, bn_gamma, bn_beta):
    row = edge_index[0]
    col = edge_index[1]
    pad = jnp.full((EP - E,), NP - 1, jnp.int32)
    row_p = jnp.concatenate([row, pad]).reshape(NTILES, NCH, CH)
    col_p = jnp.concatenate([col, pad])
    col3d = col_p.reshape(NTILES, NCH, CH)
    rc = jnp.stack([row_p, col3d], axis=2)
    zeros128 = jnp.zeros((NP, H), jnp.float32)
    ones128 = jnp.ones((CH, H), jnp.float32)

    deg_b = _deg_call(col3d, zeros128, ones128)
    y, dis, acc = _prep_call(user_emb, item_emb, deg_b)

    g = bn_gamma.reshape(LAYERS, 2, 1, H)
    b = bn_beta.reshape(LAYERS, 2, 1, H)
    nn_b = jnp.broadcast_to(
        jnp.pad(node_norm, (0, NP - N))[:, None], (NP, H))

    for i in range(LAYERS):
        s = _conv_call(y, rc, zeros128)
        if i < LAYERS - 1:
            acc, y = _bn_mid_call(s, dis, g[i], b[i], acc)
        else:
            f = _bn_final_call(s, dis, g[i], b[i], acc, nn_b)

    final = jnp.concatenate([f[0], f[1]], axis=1)
    return final[:N_USERS], final[N_USERS:N]


# speedup vs baseline: 5.3309x; 1.0617x over previous
import jax
import jax.numpy as jnp
from jax import lax
from jax.experimental import pallas as pl
from jax.experimental.pallas import tpu as pltpu
from jax.experimental.pallas import tpu_sc as plsc

N = 10000
NP = 10240
EMB = 256
H = 128
E = 160000
LAYERS = 3
EPS = 1e-5
N_USERS = 5000

NTILES = 16
CH = 128
NCH = NP // CH
EP = NTILES * NP
SLAB = NP // NTILES

def _mesh():
    return plsc.VectorSubcoreMesh(core_axis_name="c", subcore_axis_name="s")


NCHD = NCH // 2


def _deg_body(col_hbm, zeros_hbm, ones_hbm, deg_hbm, deg_sh, colb, ones,
              s0, s1, s2, s3):
    cid = lax.axis_index("c")
    tid = lax.axis_index("s")
    ssem = (s0, s1, s2, s3)

    slab = pl.ds(tid * SLAB, SLAB)
    pltpu.sync_copy(zeros_hbm.at[slab], deg_sh.at[slab])
    pltpu.sync_copy(ones_hbm, ones)
    pltpu.sync_copy(col_hbm.at[cid].at[tid], colb)
    plsc.subcore_barrier()

    @pl.loop(0, NCHD, step=4)
    def _(k0):
        for b in range(4):
            k = k0 + b

            @pl.when(k >= 4)
            def _():
                pltpu.make_async_copy(
                    ones, deg_sh.at[colb.at[k - 4]], ssem[b]).wait()
            pltpu.make_async_copy(
                ones, deg_sh.at[colb.at[k]], ssem[b]).start(add=True)

    for b in range(4):
        pltpu.make_async_copy(
            ones, deg_sh.at[colb.at[NCHD - 4 + b]], ssem[b]).wait()

    plsc.subcore_barrier()
    pltpu.sync_copy(deg_sh.at[slab], deg_hbm.at[cid].at[slab])


def _deg_call(col4d, zeros128, ones128):
    return pl.kernel(
        _deg_body,
        out_type=jax.ShapeDtypeStruct((2, NP, H), jnp.float32),
        mesh=_mesh(),
        scratch_types=[
            pltpu.VMEM_SHARED((NP, H), jnp.float32),
            pltpu.VMEM((NCHD, CH), jnp.int32),
            pltpu.VMEM((CH, H), jnp.float32),
            pltpu.SemaphoreType.DMA,
            pltpu.SemaphoreType.DMA,
            pltpu.SemaphoreType.DMA,
            pltpu.SemaphoreType.DMA,
        ],
    )(col4d, zeros128, ones128)


def _conv_body(y_hbm, rc_hbm, zeros_hbm, s_hbm,
               acc_sh, rc0, rc1, m0, m1,
               i0, i1, g0, g1, s0, s1):
    cid = lax.axis_index("c")
    tid = lax.axis_index("s")
    slab = pl.ds(tid * SLAB, SLAB)
    rcb = (rc0, rc1)
    msgs = (m0, m1)
    isem = (i0, i1)
    gsem = (g0, g1)
    ssem = (s0, s1)

    pltpu.sync_copy(zeros_hbm.at[slab], acc_sh.at[slab])
    plsc.subcore_barrier()

    ysrc = y_hbm.at[cid]
    rcs = rc_hbm.at[tid]

    @pl.loop(0, NCH, step=2)
    def _(k0):
        for b in range(2):
            k = k0 + b
            o = b ^ 1

            @pl.when(k >= 2)
            def _():
                pltpu.make_async_copy(msgs[b], acc_sh.at[rcb[b].at[1]],
                                      ssem[b]).wait()
            pltpu.make_async_copy(rcs.at[k], rcb[b], isem[b]).start()

            @pl.when(k >= 1)
            def _():
                pltpu.make_async_copy(ysrc.at[rcb[o].at[0]], msgs[o],
                                      gsem[o]).wait()
                pltpu.make_async_copy(msgs[o], acc_sh.at[rcb[o].at[1]],
                                      ssem[o]).start(add=True)
            pltpu.make_async_copy(rcs.at[k], rcb[b], isem[b]).wait()
            pltpu.make_async_copy(ysrc.at[rcb[b].at[0]], msgs[b],
                                  gsem[b]).start()

    lb = (NCH - 1) % 2
    pltpu.make_async_copy(ysrc.at[rcb[lb].at[0]], msgs[lb], gsem[lb]).wait()
    pltpu.make_async_copy(msgs[lb], acc_sh.at[rcb[lb].at[1]],
                          ssem[lb]).start(add=True)
    pltpu.make_async_copy(msgs[0], acc_sh.at[rcb[0].at[1]], ssem[0]).wait()
    pltpu.make_async_copy(msgs[1], acc_sh.at[rcb[1].at[1]], ssem[1]).wait()

    plsc.subcore_barrier()
    pltpu.sync_copy(acc_sh.at[slab], s_hbm.at[cid].at[slab])


def _conv_call(y, rc, zeros128):
    return pl.kernel(
        _conv_body,
        out_type=jax.ShapeDtypeStruct((2, NP, H), jnp.float32),
        mesh=_mesh(),
        scratch_types=[
            pltpu.VMEM_SHARED((NP, H), jnp.float32),
            pltpu.VMEM((2, CH), jnp.int32),
            pltpu.VMEM((2, CH), jnp.int32),
            pltpu.VMEM((CH, H), jnp.float32),
            pltpu.VMEM((CH, H), jnp.float32),
            pltpu.SemaphoreType.DMA,
            pltpu.SemaphoreType.DMA,
            pltpu.SemaphoreType.DMA,
            pltpu.SemaphoreType.DMA,
            pltpu.SemaphoreType.DMA,
            pltpu.SemaphoreType.DMA,
        ],
    )(y, rc, zeros128)


def _prep_body(u_ref, i_ref, degb_ref, y_ref, dis_ref, xh_ref):
    deg = degb_ref[0] + degb_ref[1]
    rid = lax.broadcasted_iota(jnp.int32, (NP, H), 0)
    dis = jnp.where((deg > 0) & (rid < N), lax.rsqrt(deg), 0.0)
    dis_ref[...] = dis
    for c in range(2):
        xc = jnp.concatenate(
            [u_ref[:, c * H:(c + 1) * H],
             i_ref[:, c * H:(c + 1) * H],
             jnp.zeros((NP - N, H), jnp.float32)], axis=0)
        xh_ref[c] = xc
        y_ref[c] = dis * xc


def _prep_call(user_emb, item_emb, deg_b):
    return pl.pallas_call(
        _prep_body,
        out_shape=(
            jax.ShapeDtypeStruct((2, NP, H), jnp.float32),
            jax.ShapeDtypeStruct((NP, H), jnp.float32),
            jax.ShapeDtypeStruct((2, NP, H), jnp.float32),
        ),
    )(user_emb, item_emb, deg_b)


def _bn_mid_body(s_ref, dis_ref, g_ref, b_ref, acc_ref, accout_ref, y_ref):
    dis = dis_ref[...]
    inv_n = jnp.float32(1.0 / N)
    for c in range(2):
        h = dis * s_ref[c]
        mean = jnp.sum(h, axis=0, keepdims=True) * inv_n
        var = jnp.sum(h * h, axis=0, keepdims=True) * inv_n - mean * mean
        hr = jnp.maximum(
            (h - mean) * lax.rsqrt(var + EPS) * g_ref[c] + b_ref[c], 0.0)
        accout_ref[c] = acc_ref[c] + hr
        y_ref[c] = dis * hr


def _bn_mid_call(s, dis, g, b, acc):
    return pl.pallas_call(
        _bn_mid_body,
        out_shape=(
            jax.ShapeDtypeStruct((2, NP, H), jnp.float32),
            jax.ShapeDtypeStruct((2, NP, H), jnp.float32),
        ),
    )(s, dis, g, b, acc)


def _bn_final_body(s_ref, dis_ref, g_ref, b_ref, acc_ref, nn_ref, f_ref):
    dis = dis_ref[...]
    nn = nn_ref[...]
    inv_n = jnp.float32(1.0 / N)
    for c in range(2):
        h = dis * s_ref[c]
        mean = jnp.sum(h, axis=0, keepdims=True) * inv_n
        var = jnp.sum(h * h, axis=0, keepdims=True) * inv_n - mean * mean
        hr = jnp.maximum(
            (h - mean) * lax.rsqrt(var + EPS) * g_ref[c] + b_ref[c], 0.0)
        f_ref[c] = (acc_ref[c] + hr) * 0.25 * nn


def _bn_final_call(s, dis, g, b, acc, nn_b):
    return pl.pallas_call(
        _bn_final_body,
        out_shape=jax.ShapeDtypeStruct((2, NP, H), jnp.float32),
    )(s, dis, g, b, acc, nn_b)


def kernel(edge_index, node_norm, user_emb, item_emb, bn_gamma, bn_beta):
    row = edge_index[0]
    col = edge_index[1]
    pad = jnp.full((EP - E,), NP - 1, jnp.int32)
    row_p = jnp.concatenate([row, pad]).reshape(NTILES, NCH, CH)
    col_p = jnp.concatenate([col, pad])
    rc = jnp.stack([row_p, col_p.reshape(NTILES, NCH, CH)], axis=2)
    col4d = col_p.reshape(2, NTILES, NCHD, CH)
    zeros128 = jnp.zeros((NP, H), jnp.float32)
    ones128 = jnp.ones((CH, H), jnp.float32)

    deg_b = _deg_call(col4d, zeros128, ones128)
    y, dis, acc = _prep_call(user_emb, item_emb, deg_b)

    g = bn_gamma.reshape(LAYERS, 2, 1, H)
    b = bn_beta.reshape(LAYERS, 2, 1, H)
    nn_b = jnp.broadcast_to(
        jnp.pad(node_norm, (0, NP - N))[:, None], (NP, H))

    for i in range(LAYERS):
        s = _conv_call(y, rc, zeros128)
        if i < LAYERS - 1:
            acc, y = _bn_mid_call(s, dis, g[i], b[i], acc)
        else:
            f = _bn_final_call(s, dis, g[i], b[i], acc, nn_b)

    final = jnp.concatenate([f[0], f[1]], axis=1)
    return final[:N_USERS], final[N_USERS:N]


# speedup vs baseline: 5.3459x; 1.0028x over previous
import jax
import jax.numpy as jnp
from jax import lax
from jax.experimental import pallas as pl
from jax.experimental.pallas import tpu as pltpu
from jax.experimental.pallas import tpu_sc as plsc

N = 10000
NP = 10240
EMB = 256
H = 128
E = 160000
LAYERS = 3
EPS = 1e-5
N_USERS = 5000

NTILES = 16
CH = 128
NCH = NP // CH
EP = NTILES * NP
SLAB = NP // NTILES

def _mesh():
    return plsc.VectorSubcoreMesh(core_axis_name="c", subcore_axis_name="s")


NCHD = NCH // 2


def _deg_body(col_hbm, zeros_hbm, ones_hbm, deg_hbm, deg_sh, colb, ones,
              s0, s1, s2, s3):
    cid = lax.axis_index("c")
    tid = lax.axis_index("s")
    ssem = (s0, s1, s2, s3)

    slab = pl.ds(tid * SLAB, SLAB)
    pltpu.sync_copy(zeros_hbm.at[slab], deg_sh.at[slab])
    pltpu.sync_copy(ones_hbm, ones)
    pltpu.sync_copy(col_hbm.at[cid].at[tid], colb)
    plsc.subcore_barrier()

    @pl.loop(0, NCHD, step=4)
    def _(k0):
        for b in range(4):
            k = k0 + b

            @pl.when(k >= 4)
            def _():
                pltpu.make_async_copy(
                    ones, deg_sh.at[colb.at[k - 4]], ssem[b]).wait()
            pltpu.make_async_copy(
                ones, deg_sh.at[colb.at[k]], ssem[b]).start(add=True)

    for b in range(4):
        pltpu.make_async_copy(
            ones, deg_sh.at[colb.at[NCHD - 4 + b]], ssem[b]).wait()

    plsc.subcore_barrier()
    pltpu.sync_copy(deg_sh.at[slab], deg_hbm.at[cid].at[slab])


def _deg_call(col4d, zeros128, ones128):
    return pl.kernel(
        _deg_body,
        out_type=jax.ShapeDtypeStruct((2, NP, H), jnp.float32),
        mesh=_mesh(),
        scratch_types=[
            pltpu.VMEM_SHARED((NP, H), jnp.float32),
            pltpu.VMEM((NCHD, CH), jnp.int32),
            pltpu.VMEM((CH, H), jnp.float32),
            pltpu.SemaphoreType.DMA,
            pltpu.SemaphoreType.DMA,
            pltpu.SemaphoreType.DMA,
            pltpu.SemaphoreType.DMA,
        ],
    )(col4d, zeros128, ones128)


def _conv_body(y_hbm, rc_hbm, zeros_hbm, s_hbm,
               acc_sh, rc0, rc1, m0, m1,
               i0, i1, g0, g1, s0, s1):
    cid = lax.axis_index("c")
    tid = lax.axis_index("s")
    slab = pl.ds(tid * SLAB, SLAB)
    rcb = (rc0, rc1)
    msgs = (m0, m1)
    isem = (i0, i1)
    gsem = (g0, g1)
    ssem = (s0, s1)

    pltpu.sync_copy(zeros_hbm.at[slab], acc_sh.at[slab])
    plsc.subcore_barrier()

    ysrc = y_hbm.at[cid]
    rcs = rc_hbm.at[tid]

    @pl.loop(0, NCH, step=2)
    def _(k0):
        for b in range(2):
            k = k0 + b
            o = b ^ 1

            @pl.when(k >= 2)
            def _():
                pltpu.make_async_copy(msgs[b], acc_sh.at[rcb[b].at[1]],
                                      ssem[b]).wait()
            pltpu.make_async_copy(rcs.at[k], rcb[b], isem[b]).start()

            @pl.when(k >= 1)
            def _():
                pltpu.make_async_copy(ysrc.at[rcb[o].at[0]], msgs[o],
                                      gsem[o]).wait()
                pltpu.make_async_copy(msgs[o], acc_sh.at[rcb[o].at[1]],
                                      ssem[o]).start(add=True)
            pltpu.make_async_copy(rcs.at[k], rcb[b], isem[b]).wait()
            pltpu.make_async_copy(ysrc.at[rcb[b].at[0]], msgs[b],
                                  gsem[b]).start()

    lb = (NCH - 1) % 2
    pltpu.make_async_copy(ysrc.at[rcb[lb].at[0]], msgs[lb], gsem[lb]).wait()
    pltpu.make_async_copy(msgs[lb], acc_sh.at[rcb[lb].at[1]],
                          ssem[lb]).start(add=True)
    pltpu.make_async_copy(msgs[0], acc_sh.at[rcb[0].at[1]], ssem[0]).wait()
    pltpu.make_async_copy(msgs[1], acc_sh.at[rcb[1].at[1]], ssem[1]).wait()

    plsc.subcore_barrier()
    pltpu.sync_copy(acc_sh.at[slab], s_hbm.at[cid].at[slab])


def _conv_call(y, rc, zeros128):
    return pl.kernel(
        _conv_body,
        out_type=jax.ShapeDtypeStruct((2, NP, H), jnp.float32),
        mesh=_mesh(),
        scratch_types=[
            pltpu.VMEM_SHARED((NP, H), jnp.float32),
            pltpu.VMEM((2, CH), jnp.int32),
            pltpu.VMEM((2, CH), jnp.int32),
            pltpu.VMEM((CH, H), jnp.float32),
            pltpu.VMEM((CH, H), jnp.float32),
            pltpu.SemaphoreType.DMA,
            pltpu.SemaphoreType.DMA,
            pltpu.SemaphoreType.DMA,
            pltpu.SemaphoreType.DMA,
            pltpu.SemaphoreType.DMA,
            pltpu.SemaphoreType.DMA,
        ],
    )(y, rc, zeros128)


def _prep_body(u_ref, i_ref, degb_ref, y_ref, dis_ref, xh_ref):
    deg = degb_ref[0] + degb_ref[1]
    rid = lax.broadcasted_iota(jnp.int32, (NP, H), 0)
    dis = jnp.where((deg > 0) & (rid < N), lax.rsqrt(deg), 0.0)
    dis_ref[...] = dis
    for c in range(2):
        xc = jnp.concatenate(
            [u_ref[:, c * H:(c + 1) * H],
             i_ref[:, c * H:(c + 1) * H],
             jnp.zeros((NP - N, H), jnp.float32)], axis=0)
        xh_ref[c] = xc
        y_ref[c] = dis * xc


def _prep_call(user_emb, item_emb, deg_b):
    return pl.pallas_call(
        _prep_body,
        out_shape=(
            jax.ShapeDtypeStruct((2, NP, H), jnp.float32),
            jax.ShapeDtypeStruct((NP, H), jnp.float32),
            jax.ShapeDtypeStruct((2, NP, H), jnp.float32),
        ),
    )(user_emb, item_emb, deg_b)


def _bn_mid_body(s_ref, dis_ref, g_ref, b_ref, acc_ref, accout_ref, y_ref):
    dis = dis_ref[...]
    inv_n = jnp.float32(1.0 / N)
    for c in range(2):
        h = dis * s_ref[c]
        mean = jnp.sum(h, axis=0, keepdims=True) * inv_n
        var = jnp.sum(h * h, axis=0, keepdims=True) * inv_n - mean * mean
        hr = jnp.maximum(
            (h - mean) * lax.rsqrt(var + EPS) * g_ref[c] + b_ref[c], 0.0)
        accout_ref[c] = acc_ref[c] + hr
        y_ref[c] = dis * hr


def _bn_mid_call(s, dis, g, b, acc):
    return pl.pallas_call(
        _bn_mid_body,
        out_shape=(
            jax.ShapeDtypeStruct((2, NP, H), jnp.float32),
            jax.ShapeDtypeStruct((2, NP, H), jnp.float32),
        ),
    )(s, dis, g, b, acc)


def _bn_final_body(s_ref, dis_ref, g_ref, b_ref, acc_ref, nn_ref,
                   u_ref, i_ref):
    dis = dis_ref[...]
    nn = nn_ref[...]
    inv_n = jnp.float32(1.0 / N)
    for c in range(2):
        h = dis * s_ref[c]
        mean = jnp.sum(h, axis=0, keepdims=True) * inv_n
        var = jnp.sum(h * h, axis=0, keepdims=True) * inv_n - mean * mean
        hr = jnp.maximum(
            (h - mean) * lax.rsqrt(var + EPS) * g_ref[c] + b_ref[c], 0.0)
        f = (acc_ref[c] + hr) * 0.25 * nn
        u_ref[:, c * H:(c + 1) * H] = f[:N_USERS]
        i_ref[:, c * H:(c + 1) * H] = f[N_USERS:N]


def _bn_final_call(s, dis, g, b, acc, nn_b):
    return pl.pallas_call(
        _bn_final_body,
        out_shape=(
            jax.ShapeDtypeStruct((N_USERS, EMB), jnp.float32),
            jax.ShapeDtypeStruct((N - N_USERS, EMB), jnp.float32),
        ),
    )(s, dis, g, b, acc, nn_b)


def kernel(edge_index, node_norm, user_emb, item_emb, bn_gamma, bn_beta):
    row = edge_index[0]
    col = edge_index[1]
    pad = jnp.full((EP - E,), NP - 1, jnp.int32)
    row_p = jnp.concatenate([row, pad]).reshape(NTILES, NCH, CH)
    col_p = jnp.concatenate([col, pad])
    rc = jnp.stack([row_p, col_p.reshape(NTILES, NCH, CH)], axis=2)
    col4d = col_p.reshape(2, NTILES, NCHD, CH)
    zeros128 = jnp.zeros((NP, H), jnp.float32)
    ones128 = jnp.ones((CH, H), jnp.float32)

    deg_b = _deg_call(col4d, zeros128, ones128)
    y, dis, acc = _prep_call(user_emb, item_emb, deg_b)

    g = bn_gamma.reshape(LAYERS, 2, 1, H)
    b = bn_beta.reshape(LAYERS, 2, 1, H)
    nn_b = jnp.broadcast_to(
        jnp.pad(node_norm, (0, NP - N))[:, None], (NP, H))

    for i in range(LAYERS):
        s = _conv_call(y, rc, zeros128)
        if i < LAYERS - 1:
            acc, y = _bn_mid_call(s, dis, g[i], b[i], acc)
        else:
            u_final, i_final = _bn_final_call(s, dis, g[i], b[i], acc, nn_b)

    return u_final, i_final
